# baseline (speedup 1.0000x reference)
import functools

import jax
import jax.numpy as jnp
from jax import lax
from jax.experimental import pallas as pl
from jax.experimental.pallas import tpu as pltpu
from jax.experimental.pallas import tpu_sc as plsc

_TOKENS = 32768
_D = 768
_E = 64
_BT = 1024

_info = plsc.get_sparse_core_info()
_NC = _info.num_cores
_NS = _info.num_subcores
_NW = _NC * _NS
_RPW = _TOKENS // _NW
_TILE = 128
_NTILE = _RPW // _TILE
_LOW6 = -64


def _probs_body(x_ref, w_ref, b_ref, probs_ref):
    logits = jnp.dot(x_ref[...], w_ref[...], preferred_element_type=jnp.float32)
    logits = logits + b_ref[...]
    m = jnp.max(logits, axis=1, keepdims=True)
    e = jnp.exp(logits - m)
    probs_ref[...] = e / jnp.sum(e, axis=1, keepdims=True)


def _topk_body(probs_hbm, w1_hbm, w2_hbm, i1_hbm, i2_hbm,
               buf_a, buf_b, w1b, w2b, i1b, i2b, sem_a, sem_b):
    c = lax.axis_index("c")
    s = lax.axis_index("s")
    wid = s * _NC + c
    base = wid * _RPW
    bufs = (buf_a, buf_b)
    sems = (sem_a, sem_b)
    lane = lax.broadcasted_iota(jnp.int32, (16,), 0)

    copies = [None, None]
    copies[0] = pltpu.async_copy(
        probs_hbm.at[pl.ds(base * _E, _TILE * _E)], buf_a, sem_a)
    for t in range(_NTILE):
        if t + 1 < _NTILE:
            copies[(t + 1) % 2] = pltpu.async_copy(
                probs_hbm.at[pl.ds((base + (t + 1) * _TILE) * _E, _TILE * _E)],
                bufs[(t + 1) % 2], sems[(t + 1) % 2])
        copies[t % 2].wait()
        buf = bufs[t % 2]

        def rg_body(rg, _, buf=buf, t=t):
            fbase = (rg * 16 + lane) * _E
            tops = []
            for c0 in range(4):
                m1 = jnp.full((16,), -1, jnp.int32)
                m2 = jnp.full((16,), -1, jnp.int32)
                for e in range(c0 * 16, c0 * 16 + 16):
                    v = plsc.load_gather(buf, [fbase + e])
                    p = (plsc.bitcast(v, jnp.int32) & _LOW6) | (63 - e)
                    gt1 = p > m1
                    m2 = jnp.where(gt1, m1, jnp.maximum(p, m2))
                    m1 = jnp.where(gt1, p, m1)
                tops.append((m1, m2))

            def merge(a, b):
                hi = jnp.maximum(a[0], b[0])
                lo = jnp.minimum(a[0], b[0])
                return hi, jnp.maximum(lo, jnp.maximum(a[1], b[1]))

            m1, m2 = merge(merge(tops[0], tops[1]), merge(tops[2], tops[3]))
            i1 = 63 - (m1 & 63)
            i2 = 63 - (m2 & 63)
            v1 = plsc.load_gather(buf, [fbase + i1])
            v2 = plsc.load_gather(buf, [fbase + i2])
            ws = v1 + v2
            off = t * _TILE + rg * 16
            w1b[pl.ds(off, 16)] = v1 / ws
            w2b[pl.ds(off, 16)] = v2 / ws
            i1b[pl.ds(off, 16)] = i1
            i2b[pl.ds(off, 16)] = i2
            return 0

        lax.fori_loop(0, _TILE // 16, rg_body, 0)

    pltpu.sync_copy(w1b, w1_hbm.at[pl.ds(base, _RPW)])
    pltpu.sync_copy(w2b, w2_hbm.at[pl.ds(base, _RPW)])
    pltpu.sync_copy(i1b, i1_hbm.at[pl.ds(base, _RPW)])
    pltpu.sync_copy(i2b, i2_hbm.at[pl.ds(base, _RPW)])


_topk_sc = functools.partial(
    pl.kernel,
    out_type=[
        jax.ShapeDtypeStruct((_TOKENS,), jnp.float32),
        jax.ShapeDtypeStruct((_TOKENS,), jnp.float32),
        jax.ShapeDtypeStruct((_TOKENS,), jnp.int32),
        jax.ShapeDtypeStruct((_TOKENS,), jnp.int32),
    ],
    mesh=plsc.VectorSubcoreMesh(core_axis_name="c", subcore_axis_name="s"),
    compiler_params=pltpu.CompilerParams(needs_layout_passes=False),
    scratch_types=[
        pltpu.VMEM((_TILE * _E,), jnp.float32),
        pltpu.VMEM((_TILE * _E,), jnp.float32),
        pltpu.VMEM((_RPW,), jnp.float32),
        pltpu.VMEM((_RPW,), jnp.float32),
        pltpu.VMEM((_RPW,), jnp.int32),
        pltpu.VMEM((_RPW,), jnp.int32),
        pltpu.SemaphoreType.DMA,
        pltpu.SemaphoreType.DMA,
    ],
)(_topk_body)


@jax.jit
def kernel(x, W, b):
    grid = _TOKENS // _BT
    probs = pl.pallas_call(
        _probs_body,
        grid=(grid,),
        in_specs=[
            pl.BlockSpec((_BT, _D), lambda i: (i, 0)),
            pl.BlockSpec((_D, _E), lambda i: (0, 0)),
            pl.BlockSpec((1, _E), lambda i: (0, 0)),
        ],
        out_specs=pl.BlockSpec((_BT, _E), lambda i: (i, 0)),
        out_shape=jax.ShapeDtypeStruct((_TOKENS, _E), jnp.float32),
    )(x, W, b.reshape(1, _E))
    w1, w2, i1, i2 = _topk_sc(probs.reshape(-1))
    tw = jnp.stack([w1, w2], axis=1)
    ti = jnp.stack([i1, i2], axis=1)
    return tw, ti, probs


# speedup vs baseline: 1.0174x; 1.0174x over previous
import functools

import jax
import jax.numpy as jnp
from jax import lax
from jax.experimental import pallas as pl
from jax.experimental.pallas import tpu as pltpu
from jax.experimental.pallas import tpu_sc as plsc

_TOKENS = 32768
_D = 768
_E = 64
_BT = 1024

_info = plsc.get_sparse_core_info()
_NC = _info.num_cores
_NS = _info.num_subcores
_NW = _NC * _NS
_RPW = _TOKENS // _NW
_TILE = 128
_NTILE = _RPW // _TILE


def _probs_body(x_ref, w_ref, b_ref, probs_ref):
    logits = jnp.dot(x_ref[...], w_ref[...], preferred_element_type=jnp.float32)
    logits = logits + b_ref[...]
    m = jnp.max(logits, axis=1, keepdims=True)
    e = jnp.exp(logits - m)
    probs_ref[...] = e / jnp.sum(e, axis=1, keepdims=True)


def _topk_body(probs_hbm, w1_hbm, w2_hbm, i1_hbm, i2_hbm,
               buf_a, buf_b, w1b, w2b, i1b, i2b, sem_a, sem_b):
    c = lax.axis_index("c")
    s = lax.axis_index("s")
    wid = s * _NC + c
    base = wid * _RPW
    bufs = (buf_a, buf_b)
    sems = (sem_a, sem_b)
    lane = lax.broadcasted_iota(jnp.int32, (16,), 0)

    copies = [None, None]
    copies[0] = pltpu.async_copy(
        probs_hbm.at[pl.ds(base * _E, _TILE * _E)], buf_a, sem_a)
    for t in range(_NTILE):
        if t + 1 < _NTILE:
            copies[(t + 1) % 2] = pltpu.async_copy(
                probs_hbm.at[pl.ds((base + (t + 1) * _TILE) * _E, _TILE * _E)],
                bufs[(t + 1) % 2], sems[(t + 1) % 2])
        copies[t % 2].wait()
        buf = bufs[t % 2]

        def rg_body(rg, _, buf=buf, t=t):
            fbase = (rg * 16 + lane) * _E
            tops = []
            for c0 in range(4):
                m1 = jnp.full((16,), -1.0, jnp.float32)
                m2 = jnp.full((16,), -1.0, jnp.float32)
                i1 = jnp.full((16,), 0, jnp.int32)
                i2 = jnp.full((16,), 0, jnp.int32)
                for e in range(c0 * 16, c0 * 16 + 16):
                    v = plsc.load_gather(buf, [fbase + e])
                    gt1 = v > m1
                    gt2 = v > m2
                    m2 = jnp.where(gt1, m1, jnp.where(gt2, v, m2))
                    i2 = jnp.where(gt1, i1, jnp.where(gt2, e, i2))
                    m1 = jnp.where(gt1, v, m1)
                    i1 = jnp.where(gt1, e, i1)
                tops.append((m1, i1, m2, i2))

            def merge(a, b):
                a1, ai1, a2, ai2 = a
                b1, bi1, b2, bi2 = b
                a_wins = a1 >= b1
                hi = jnp.where(a_wins, a1, b1)
                hij = jnp.where(a_wins, ai1, bi1)
                lo = jnp.where(a_wins, b1, a1)
                loj = jnp.where(a_wins, bi1, ai1)
                s2 = jnp.where(a_wins, a2, b2)
                s2j = jnp.where(a_wins, ai2, bi2)
                sec_w = s2 > lo
                sec = jnp.where(sec_w, s2, lo)
                secj = jnp.where(sec_w, s2j, loj)
                return hi, hij, sec, secj

            m1, i1, m2, i2 = merge(merge(tops[0], tops[1]),
                                   merge(tops[2], tops[3]))
            v1 = m1
            v2 = m2
            ws = v1 + v2
            off = t * _TILE + rg * 16
            w1b[pl.ds(off, 16)] = v1 / ws
            w2b[pl.ds(off, 16)] = v2 / ws
            i1b[pl.ds(off, 16)] = i1
            i2b[pl.ds(off, 16)] = i2
            return 0

        lax.fori_loop(0, _TILE // 16, rg_body, 0)

    pltpu.sync_copy(w1b, w1_hbm.at[pl.ds(base, _RPW)])
    pltpu.sync_copy(w2b, w2_hbm.at[pl.ds(base, _RPW)])
    pltpu.sync_copy(i1b, i1_hbm.at[pl.ds(base, _RPW)])
    pltpu.sync_copy(i2b, i2_hbm.at[pl.ds(base, _RPW)])


_topk_sc = functools.partial(
    pl.kernel,
    out_type=[
        jax.ShapeDtypeStruct((_TOKENS,), jnp.float32),
        jax.ShapeDtypeStruct((_TOKENS,), jnp.float32),
        jax.ShapeDtypeStruct((_TOKENS,), jnp.int32),
        jax.ShapeDtypeStruct((_TOKENS,), jnp.int32),
    ],
    mesh=plsc.VectorSubcoreMesh(core_axis_name="c", subcore_axis_name="s"),
    compiler_params=pltpu.CompilerParams(needs_layout_passes=False),
    scratch_types=[
        pltpu.VMEM((_TILE * _E,), jnp.float32),
        pltpu.VMEM((_TILE * _E,), jnp.float32),
        pltpu.VMEM((_RPW,), jnp.float32),
        pltpu.VMEM((_RPW,), jnp.float32),
        pltpu.VMEM((_RPW,), jnp.int32),
        pltpu.VMEM((_RPW,), jnp.int32),
        pltpu.SemaphoreType.DMA,
        pltpu.SemaphoreType.DMA,
    ],
)(_topk_body)


@jax.jit
def kernel(x, W, b):
    grid = _TOKENS // _BT
    probs = pl.pallas_call(
        _probs_body,
        grid=(grid,),
        in_specs=[
            pl.BlockSpec((_BT, _D), lambda i: (i, 0)),
            pl.BlockSpec((_D, _E), lambda i: (0, 0)),
            pl.BlockSpec((1, _E), lambda i: (0, 0)),
        ],
        out_specs=pl.BlockSpec((_BT, _E), lambda i: (i, 0)),
        out_shape=jax.ShapeDtypeStruct((_TOKENS, _E), jnp.float32),
    )(x, W, b.reshape(1, _E))
    w1, w2, i1, i2 = _topk_sc(probs.reshape(-1))
    tw = jnp.stack([w1, w2], axis=1)
    ti = jnp.stack([i1, i2], axis=1)
    return tw, ti, probs


# speedup vs baseline: 1.2021x; 1.1815x over previous
import functools

import jax
import jax.numpy as jnp
from jax import lax
from jax.experimental import pallas as pl
from jax.experimental.pallas import tpu as pltpu
from jax.experimental.pallas import tpu_sc as plsc

_TOKENS = 32768
_D = 768
_E = 64
_BT = 4096

_info = plsc.get_sparse_core_info()
_NC = _info.num_cores
_NS = _info.num_subcores
_NW = _NC * _NS
_RPW = _TOKENS // _NW
_TILE = 256
_NTILE = _RPW // _TILE


def _probs_body(x_ref, w_ref, b_ref, probs_ref):
    logits = jnp.dot(x_ref[...], w_ref[...], preferred_element_type=jnp.float32)
    logits = logits + b_ref[...]
    m = jnp.max(logits, axis=1, keepdims=True)
    e = jnp.exp(logits - m)
    probs_ref[...] = e / jnp.sum(e, axis=1, keepdims=True)


def _make_topk_body(tok_off, rpw):
    ntile = rpw // _TILE

    def _topk_body(probs_hbm, w1_hbm, w2_hbm, i1_hbm, i2_hbm,
                   buf_a, buf_b, w1b, w2b, i1b, i2b, sem_a, sem_b):
        c = lax.axis_index("c")
        s = lax.axis_index("s")
        wid = s * _NC + c
        base = wid * rpw
        src_base = tok_off + base
        bufs = (buf_a, buf_b)
        sems = (sem_a, sem_b)
        lane = lax.broadcasted_iota(jnp.int32, (16,), 0)

        copies = [None, None]
        copies[0] = pltpu.async_copy(
            probs_hbm.at[pl.ds(src_base * _E, _TILE * _E)], buf_a, sem_a)
        for t in range(ntile):
            if t + 1 < ntile:
                copies[(t + 1) % 2] = pltpu.async_copy(
                    probs_hbm.at[
                        pl.ds((src_base + (t + 1) * _TILE) * _E, _TILE * _E)],
                    bufs[(t + 1) % 2], sems[(t + 1) % 2])
            copies[t % 2].wait()
            buf = bufs[t % 2]

            def rg_body(rg2, _, buf=buf, t=t):
                outs = []
                for h in range(2):
                    rg = rg2 * 2 + h
                    fbase = (rg * 16 + lane) * _E
                    tops = []
                    for c0 in range(4):
                        m1 = jnp.full((16,), -1.0, jnp.float32)
                        m2 = jnp.full((16,), -1.0, jnp.float32)
                        i1 = jnp.full((16,), 0, jnp.int32)
                        i2 = jnp.full((16,), 0, jnp.int32)
                        for k in range(c0 * 16, c0 * 16 + 16):
                            e_vec = (lane + k) & 63
                            v = plsc.load_gather(buf, [fbase + e_vec])
                            gt1 = v > m1
                            gt2 = v > m2
                            m2 = jnp.where(gt1, m1, jnp.where(gt2, v, m2))
                            i2 = jnp.where(gt1, i1, jnp.where(gt2, e_vec, i2))
                            m1 = jnp.where(gt1, v, m1)
                            i1 = jnp.where(gt1, e_vec, i1)
                        tops.append((m1, i1, m2, i2))

                    def merge(a, b):
                        a1, ai1, a2, ai2 = a
                        b1, bi1, b2, bi2 = b
                        a_wins = a1 >= b1
                        hi = jnp.where(a_wins, a1, b1)
                        hij = jnp.where(a_wins, ai1, bi1)
                        lo = jnp.where(a_wins, b1, a1)
                        loj = jnp.where(a_wins, bi1, ai1)
                        s2 = jnp.where(a_wins, a2, b2)
                        s2j = jnp.where(a_wins, ai2, bi2)
                        sec_w = s2 > lo
                        sec = jnp.where(sec_w, s2, lo)
                        secj = jnp.where(sec_w, s2j, loj)
                        return hi, hij, sec, secj

                    m1, i1, m2, i2 = merge(merge(tops[0], tops[1]),
                                           merge(tops[2], tops[3]))
                    ws = m1 + m2
                    off = t * _TILE + rg * 16
                    outs.append((off, m1 / ws, m2 / ws, i1, i2))
                for off, w1v, w2v, i1v, i2v in outs:
                    w1b[pl.ds(off, 16)] = w1v
                    w2b[pl.ds(off, 16)] = w2v
                    i1b[pl.ds(off, 16)] = i1v
                    i2b[pl.ds(off, 16)] = i2v
                return 0

            lax.fori_loop(0, _TILE // 32, rg_body, 0)

        pltpu.sync_copy(w1b, w1_hbm.at[pl.ds(base, rpw)])
        pltpu.sync_copy(w2b, w2_hbm.at[pl.ds(base, rpw)])
        pltpu.sync_copy(i1b, i1_hbm.at[pl.ds(base, rpw)])
        pltpu.sync_copy(i2b, i2_hbm.at[pl.ds(base, rpw)])

    return _topk_body


_CHUNKS = 2
_CTOK = _TOKENS // _CHUNKS
_RPWC = _CTOK // _NW


def _make_topk_sc(tok_off):
    return functools.partial(
        pl.kernel,
        out_type=[
            jax.ShapeDtypeStruct((_CTOK,), jnp.float32),
            jax.ShapeDtypeStruct((_CTOK,), jnp.float32),
            jax.ShapeDtypeStruct((_CTOK,), jnp.int32),
            jax.ShapeDtypeStruct((_CTOK,), jnp.int32),
        ],
        mesh=plsc.VectorSubcoreMesh(core_axis_name="c", subcore_axis_name="s"),
        compiler_params=pltpu.CompilerParams(needs_layout_passes=False),
        scratch_types=[
            pltpu.VMEM((_TILE * _E,), jnp.float32),
            pltpu.VMEM((_TILE * _E,), jnp.float32),
            pltpu.VMEM((_RPWC,), jnp.float32),
            pltpu.VMEM((_RPWC,), jnp.float32),
            pltpu.VMEM((_RPWC,), jnp.int32),
            pltpu.VMEM((_RPWC,), jnp.int32),
            pltpu.SemaphoreType.DMA,
            pltpu.SemaphoreType.DMA,
        ],
    )(_make_topk_body(tok_off, _RPWC))


_topk_sc_a = _make_topk_sc(0)
_topk_sc_b = _make_topk_sc(_CTOK)


def _probs_body_dup(x_ref, w_ref, b_ref, probs_ref, dup_ref):
    logits = jnp.dot(x_ref[...], w_ref[...], preferred_element_type=jnp.float32)
    logits = logits + b_ref[...]
    m = jnp.max(logits, axis=1, keepdims=True)
    e = jnp.exp(logits - m)
    p = e / jnp.sum(e, axis=1, keepdims=True)
    probs_ref[...] = p
    dup_ref[...] = p


def _probs_body_tail(x_ref, w_ref, b_ref, probs_in_ref, probs_ref):
    del probs_in_ref
    logits = jnp.dot(x_ref[...], w_ref[...], preferred_element_type=jnp.float32)
    logits = logits + b_ref[...]
    m = jnp.max(logits, axis=1, keepdims=True)
    e = jnp.exp(logits - m)
    probs_ref[...] = e / jnp.sum(e, axis=1, keepdims=True)


@jax.jit
def kernel(x, W, b):
    b2 = b.reshape(1, _E)
    grid = _CTOK // _BT
    probs0, probs_a = pl.pallas_call(
        _probs_body_dup,
        grid=(grid,),
        in_specs=[
            pl.BlockSpec((_BT, _D), lambda i: (i, 0)),
            pl.BlockSpec((_D, _E), lambda i: (0, 0)),
            pl.BlockSpec((1, _E), lambda i: (0, 0)),
        ],
        out_specs=[
            pl.BlockSpec((_BT, _E), lambda i: (i, 0)),
            pl.BlockSpec((_BT, _E), lambda i: (i, 0)),
        ],
        out_shape=[
            jax.ShapeDtypeStruct((_TOKENS, _E), jnp.float32),
            jax.ShapeDtypeStruct((_CTOK, _E), jnp.float32),
        ],
    )(x, W, b2)
    probs = pl.pallas_call(
        _probs_body_tail,
        grid=(grid,),
        in_specs=[
            pl.BlockSpec((_BT, _D), lambda i: (i + grid, 0)),
            pl.BlockSpec((_D, _E), lambda i: (0, 0)),
            pl.BlockSpec((1, _E), lambda i: (0, 0)),
            pl.BlockSpec(memory_space=pl.ANY),
        ],
        out_specs=pl.BlockSpec((_BT, _E), lambda i: (i + grid, 0)),
        out_shape=jax.ShapeDtypeStruct((_TOKENS, _E), jnp.float32),
        input_output_aliases={3: 0},
    )(x, W, b2, probs0)
    oa = _topk_sc_a(probs_a.reshape(-1))
    ob = _topk_sc_b(probs.reshape(-1))
    w1 = jnp.concatenate([oa[0], ob[0]])
    w2 = jnp.concatenate([oa[1], ob[1]])
    i1 = jnp.concatenate([oa[2], ob[2]])
    i2 = jnp.concatenate([oa[3], ob[3]])
    tw = jnp.stack([w1, w2], axis=1)
    ti = jnp.stack([i1, i2], axis=1)
    return tw, ti, probs


# speedup vs baseline: 1.3473x; 1.1208x over previous
import functools

import jax
import jax.numpy as jnp
from jax import lax
from jax.experimental import pallas as pl
from jax.experimental.pallas import tpu as pltpu
from jax.experimental.pallas import tpu_sc as plsc

_TOKENS = 32768
_D = 768
_E = 64
_BT = 4096

_info = plsc.get_sparse_core_info()
_NC = _info.num_cores
_NS = _info.num_subcores
_NW = _NC * _NS
_RPW = _TOKENS // _NW
_TILE = 256
_NTILE = _RPW // _TILE


def _probs_body(x_ref, w_ref, b_ref, probs_ref):
    logits = jnp.dot(x_ref[...], w_ref[...], preferred_element_type=jnp.float32)
    logits = logits + b_ref[...]
    m = jnp.max(logits, axis=1, keepdims=True)
    e = jnp.exp(logits - m)
    probs_ref[...] = e / jnp.sum(e, axis=1, keepdims=True)


def _topk_body(probs_hbm, w1_hbm, w2_hbm, i1_hbm, i2_hbm,
               buf_a, buf_b, w1b, w2b, i1b, i2b, sem_a, sem_b):
    c = lax.axis_index("c")
    s = lax.axis_index("s")
    wid = s * _NC + c
    base = wid * _RPW
    bufs = (buf_a, buf_b)
    sems = (sem_a, sem_b)
    lane = lax.broadcasted_iota(jnp.int32, (16,), 0)

    copies = [None, None]
    copies[0] = pltpu.async_copy(
        probs_hbm.at[pl.ds(base * _E, _TILE * _E)], buf_a, sem_a)
    for t in range(_NTILE):
        if t + 1 < _NTILE:
            copies[(t + 1) % 2] = pltpu.async_copy(
                probs_hbm.at[pl.ds((base + (t + 1) * _TILE) * _E, _TILE * _E)],
                bufs[(t + 1) % 2], sems[(t + 1) % 2])
        copies[t % 2].wait()
        buf = bufs[t % 2]

        def rg_body(rg2, _, buf=buf, t=t):
            outs = []
            for h in range(2):
                rg = rg2 * 2 + h
                fbase = (rg * 16 + lane) * _E
                tops = []
                for c0 in range(4):
                    m1 = jnp.full((16,), -1.0, jnp.float32)
                    m2 = jnp.full((16,), -1.0, jnp.float32)
                    i1 = jnp.full((16,), 0, jnp.int32)
                    i2 = jnp.full((16,), 0, jnp.int32)
                    for k in range(c0 * 16, c0 * 16 + 16):
                        e_vec = (lane + k) & 63
                        v = plsc.load_gather(buf, [fbase + e_vec])
                        gt1 = v > m1
                        gt2 = v > m2
                        m2 = jnp.where(gt1, m1, jnp.where(gt2, v, m2))
                        i2 = jnp.where(gt1, i1, jnp.where(gt2, e_vec, i2))
                        m1 = jnp.where(gt1, v, m1)
                        i1 = jnp.where(gt1, e_vec, i1)
                    tops.append((m1, i1, m2, i2))

                def merge(a, b):
                    a1, ai1, a2, ai2 = a
                    b1, bi1, b2, bi2 = b
                    a_wins = a1 >= b1
                    hi = jnp.where(a_wins, a1, b1)
                    hij = jnp.where(a_wins, ai1, bi1)
                    lo = jnp.where(a_wins, b1, a1)
                    loj = jnp.where(a_wins, bi1, ai1)
                    s2 = jnp.where(a_wins, a2, b2)
                    s2j = jnp.where(a_wins, ai2, bi2)
                    sec_w = s2 > lo
                    sec = jnp.where(sec_w, s2, lo)
                    secj = jnp.where(sec_w, s2j, loj)
                    return hi, hij, sec, secj

                m1, i1, m2, i2 = merge(merge(tops[0], tops[1]),
                                       merge(tops[2], tops[3]))
                ws = m1 + m2
                off = t * _TILE + rg * 16
                outs.append((off, m1 / ws, m2 / ws, i1, i2))
            for off, w1v, w2v, i1v, i2v in outs:
                w1b[pl.ds(off, 16)] = w1v
                w2b[pl.ds(off, 16)] = w2v
                i1b[pl.ds(off, 16)] = i1v
                i2b[pl.ds(off, 16)] = i2v
            return 0

        lax.fori_loop(0, _TILE // 32, rg_body, 0)

    pltpu.sync_copy(w1b, w1_hbm.at[pl.ds(base, _RPW)])
    pltpu.sync_copy(w2b, w2_hbm.at[pl.ds(base, _RPW)])
    pltpu.sync_copy(i1b, i1_hbm.at[pl.ds(base, _RPW)])
    pltpu.sync_copy(i2b, i2_hbm.at[pl.ds(base, _RPW)])


_topk_sc = functools.partial(
    pl.kernel,
    out_type=[
        jax.ShapeDtypeStruct((_TOKENS,), jnp.float32),
        jax.ShapeDtypeStruct((_TOKENS,), jnp.float32),
        jax.ShapeDtypeStruct((_TOKENS,), jnp.int32),
        jax.ShapeDtypeStruct((_TOKENS,), jnp.int32),
    ],
    mesh=plsc.VectorSubcoreMesh(core_axis_name="c", subcore_axis_name="s"),
    compiler_params=pltpu.CompilerParams(needs_layout_passes=False),
    scratch_types=[
        pltpu.VMEM((_TILE * _E,), jnp.float32),
        pltpu.VMEM((_TILE * _E,), jnp.float32),
        pltpu.VMEM((_RPW,), jnp.float32),
        pltpu.VMEM((_RPW,), jnp.float32),
        pltpu.VMEM((_RPW,), jnp.int32),
        pltpu.VMEM((_RPW,), jnp.int32),
        pltpu.SemaphoreType.DMA,
        pltpu.SemaphoreType.DMA,
    ],
)(_topk_body)


@jax.jit
def kernel(x, W, b):
    grid = _TOKENS // _BT
    probs = pl.pallas_call(
        _probs_body,
        grid=(grid,),
        in_specs=[
            pl.BlockSpec((_BT, _D), lambda i: (i, 0)),
            pl.BlockSpec((_D, _E), lambda i: (0, 0)),
            pl.BlockSpec((1, _E), lambda i: (0, 0)),
        ],
        out_specs=pl.BlockSpec((_BT, _E), lambda i: (i, 0)),
        out_shape=jax.ShapeDtypeStruct((_TOKENS, _E), jnp.float32),
    )(x, W, b.reshape(1, _E))
    w1, w2, i1, i2 = _topk_sc(probs.reshape(-1))
    tw = jnp.stack([w1, w2], axis=1)
    ti = jnp.stack([i1, i2], axis=1)
    return tw, ti, probs
